# baseline (speedup 1.0000x reference)
import jax
import jax.numpy as jnp
from jax import lax
from jax.experimental import pallas as pl
from jax.experimental.pallas import tpu as pltpu
from jax.experimental.pallas import tpu_sc as plsc

B = 16
L = 2048
N = 16384
D = 512

NC = 2
NS = 16
NW = NC * NS
IN_PER_W = N // NW
CH = 32
CHUNKS = IN_PER_W // CH
NBUF = 7
GDEPTH = 3

_mesh = plsc.VectorSubcoreMesh(
    core_axis_name="c", subcore_axis_name="s", num_cores=NC
)


def _zeros_kernel(o_ref):
    o_ref[...] = jnp.zeros_like(o_ref)


_ZBLOCKS = 32


def _zeros_tc():
    return pl.pallas_call(
        _zeros_kernel,
        out_shape=jax.ShapeDtypeStruct((B * L, D), jnp.float32),
        grid=(_ZBLOCKS,),
        out_specs=pl.BlockSpec((B * L // _ZBLOCKS, D), lambda i: (i, 0)),
    )()


@jax.jit
def _run(feature, sample_offsets, batch_index):
    out0 = _zeros_tc()

    def stateful(refs):
        feat_hbm, off_hbm, bidx_hbm, out_hbm = refs

        @pl.core_map(
            _mesh,
            scratch_shapes=[
                [pltpu.VMEM((CH, D), jnp.float32) for _ in range(NBUF)],
                pltpu.VMEM((IN_PER_W,), jnp.int32),
                pltpu.VMEM((IN_PER_W,), jnp.int32),
                pltpu.VMEM((CHUNKS, CH), jnp.int32),
                [pltpu.SemaphoreType.DMA for _ in range(NBUF)],
                [pltpu.SemaphoreType.DMA for _ in range(NBUF)],
            ],
        )
        def _(rbufs, offf, bvf, dstv, gsems, ssems):
            wid = lax.axis_index("c") * NS + lax.axis_index("s")
            ibase = wid * IN_PER_W

            gathers = [None] * CHUNKS
            for ch in range(min(GDEPTH, CHUNKS)):
                gathers[ch] = pltpu.async_copy(
                    feat_hbm.at[pl.ds(ibase + ch * CH, CH)],
                    rbufs[ch % NBUF],
                    gsems[ch % NBUF],
                )

            pltpu.sync_copy(off_hbm.at[pl.ds(ibase, IN_PER_W)], offf)
            pltpu.sync_copy(bidx_hbm.at[pl.ds(ibase, IN_PER_W)], bvf)

            for ch in range(CHUNKS):
                @pl.loop(0, CH, step=16)
                def _(j, ch=ch):
                    s = pl.ds(ch * CH + j, 16)
                    dstv[ch, pl.ds(j, 16)] = bvf[s] * L + offf[s]

            scatters = [None] * CHUNKS
            for ch in range(CHUNKS):
                gathers[ch].wait()
                scatters[ch] = pltpu.async_copy(
                    rbufs[ch % NBUF], out_hbm.at[dstv.at[ch]], ssems[ch % NBUF]
                )
                nx = ch + GDEPTH
                if nx < CHUNKS:
                    if scatters[nx - NBUF] is not None:
                        scatters[nx - NBUF].wait()
                    gathers[nx] = pltpu.async_copy(
                        feat_hbm.at[pl.ds(ibase + nx * CH, CH)],
                        rbufs[nx % NBUF],
                        gsems[nx % NBUF],
                    )
            for ch in range(CHUNKS):
                if ch + NBUF >= CHUNKS:
                    scatters[ch].wait()

    _, _, _, out = pl.run_state(
        stateful)((feature, sample_offsets, batch_index, out0))
    return out.reshape(B, L, D)


def kernel(feature, sample_offsets, batch_index):
    return _run(feature, sample_offsets, batch_index)


# speedup vs baseline: 1.0494x; 1.0494x over previous
import jax
import jax.numpy as jnp
from jax import lax
from jax.experimental import pallas as pl
from jax.experimental.pallas import tpu as pltpu
from jax.experimental.pallas import tpu_sc as plsc

B = 16
L = 2048
N = 16384
D = 512

NC = 2
NS = 16
NW = NC * NS
IN_PER_W = N // NW
CH = 32
CHUNKS = IN_PER_W // CH
NBUF = 7
GDEPTH = 3

_mesh = plsc.VectorSubcoreMesh(
    core_axis_name="c", subcore_axis_name="s", num_cores=NC
)


def _zeros_kernel(o_ref):
    o_ref[...] = jnp.zeros_like(o_ref)


_ZBLOCKS = 16


def _zeros_tc():
    return pl.pallas_call(
        _zeros_kernel,
        out_shape=jax.ShapeDtypeStruct((B * L, D), jnp.float32),
        grid=(_ZBLOCKS,),
        out_specs=pl.BlockSpec((B * L // _ZBLOCKS, D), lambda i: (i, 0)),
    )()


@jax.jit
def _run(feature, sample_offsets, batch_index):
    out0 = _zeros_tc()

    def stateful(refs):
        feat_hbm, off_hbm, bidx_hbm, out_hbm = refs

        @pl.core_map(
            _mesh,
            scratch_shapes=[
                [pltpu.VMEM((CH, D), jnp.float32) for _ in range(NBUF)],
                pltpu.VMEM((IN_PER_W,), jnp.int32),
                pltpu.VMEM((IN_PER_W,), jnp.int32),
                pltpu.VMEM((CHUNKS, CH), jnp.int32),
                [pltpu.SemaphoreType.DMA for _ in range(NBUF)],
                [pltpu.SemaphoreType.DMA for _ in range(NBUF)],
            ],
        )
        def _(rbufs, offf, bvf, dstv, gsems, ssems):
            wid = lax.axis_index("c") * NS + lax.axis_index("s")
            ibase = wid * IN_PER_W

            gathers = [None] * CHUNKS
            for ch in range(min(GDEPTH, CHUNKS)):
                gathers[ch] = pltpu.async_copy(
                    feat_hbm.at[pl.ds(ibase + ch * CH, CH)],
                    rbufs[ch % NBUF],
                    gsems[ch % NBUF],
                )

            pltpu.sync_copy(off_hbm.at[pl.ds(ibase, IN_PER_W)], offf)
            pltpu.sync_copy(bidx_hbm.at[pl.ds(ibase, IN_PER_W)], bvf)

            for ch in range(CHUNKS):
                @pl.loop(0, CH, step=16)
                def _(j, ch=ch):
                    s = pl.ds(ch * CH + j, 16)
                    dstv[ch, pl.ds(j, 16)] = bvf[s] * L + offf[s]

            scatters = [None] * CHUNKS
            for ch in range(CHUNKS):
                gathers[ch].wait()
                scatters[ch] = pltpu.async_copy(
                    rbufs[ch % NBUF], out_hbm.at[dstv.at[ch]], ssems[ch % NBUF]
                )
                nx = ch + GDEPTH
                if nx < CHUNKS:
                    if scatters[nx - NBUF] is not None:
                        scatters[nx - NBUF].wait()
                    gathers[nx] = pltpu.async_copy(
                        feat_hbm.at[pl.ds(ibase + nx * CH, CH)],
                        rbufs[nx % NBUF],
                        gsems[nx % NBUF],
                    )
            for ch in range(CHUNKS):
                if ch + NBUF >= CHUNKS:
                    scatters[ch].wait()

    _, _, _, out = pl.run_state(
        stateful)((feature, sample_offsets, batch_index, out0))
    return out.reshape(B, L, D)


def kernel(feature, sample_offsets, batch_index):
    return _run(feature, sample_offsets, batch_index)
